# Initial kernel scaffold; baseline (speedup 1.0000x reference)
import jax
import jax.numpy as jnp
from jax.experimental import pallas as pl


def kernel(x, t, norm, flat_idx, conv_w, ln1_g, ln1_b, ln2_g, ln2_b, tmlp1_w, tmlp1_b, tmlp2_w, tmlp2_b, mlp_w1, mlp_b1, mlp_w2, mlp_b2):
    raise NotImplementedError("write your pallas kernel here")


# speedup vs baseline: 1.1270x; 1.1270x over previous
import functools
import math

import jax
import jax.numpy as jnp
from jax import lax
from jax.experimental import pallas as pl
from jax.experimental.pallas import tpu as pltpu


def _rup(v, m):
    return ((v + m - 1) // m) * m


def _gelu(v):
    return 0.5 * v * (1.0 + lax.erf(v * jnp.float32(0.7071067811865476)))


def _ln(v, g, b):
    m = jnp.mean(v, axis=-1, keepdims=True)
    c = v - m
    var = jnp.mean(c * c, axis=-1, keepdims=True)
    return c * lax.rsqrt(var + jnp.float32(1e-5)) * g + b


def _impl(H, W, x, t, norm, flat_idx, conv_w, ln1_g, ln1_b, ln2_g, ln2_b,
          tmlp1_w, tmlp1_b, tmlp2_w, tmlp2_b, mlp_w1, mlp_b1, mlp_w2, mlp_b2,
          interpret=False):
    N, C = x.shape
    B = t.shape[0]
    K = conv_w.shape[0]
    R = K // 2
    HW = H * W
    Wp = W + 2 * R
    acc_need = (H - 1) * Wp + W
    BLK = min(256, _rup(acc_need, 8))
    nblk = -(-acc_need // BLK)
    ACC_B = nblk * BLK
    omax = (K - 1) * (Wp + 1)
    PB = _rup(ACC_B + omax, 8)
    NB = N // B
    RB = 512 if NB % 512 == 0 else NB
    nrb = N // RB

    w49 = conv_w.reshape(K * K, C, C)

    def body(idx_s, x_r, t_r, norm_r, w_r, ln1g_r, ln1b_r, ln2g_r, ln2b_r,
             t1w_r, t1b_r, t2w_r, t2b_r, m1w_r, m1b_r, m2w_r, m2b_r,
             out_r, pad_s, acc_s, h_s, hc_s):
        f32 = jnp.float32
        tg = _gelu(t_r[...])
        tt1 = jnp.dot(tg, t1w_r[...], preferred_element_type=f32) + t1b_r[...]
        tt2 = jnp.dot(tg, t2w_r[...], preferred_element_type=f32) + t2b_r[...]

        ln1g = ln1g_r[...]
        ln1b = ln1b_r[...]
        for rb in range(nrb):
            sl = pl.ds(rb * RB, RB)
            bi = (rb * RB) // NB
            sc = tt1[bi:bi + 1, :C]
            sh = tt1[bi:bi + 1, C:]
            f = _ln(x_r[sl, :], ln1g, ln1b)
            h_s[sl, :] = f * (1.0 + sc) + sh

        pad_s[...] = jnp.zeros((B * PB, C), f32)

        def scat(i, c):
            v = idx_s[i]
            b = v // HW
            r = v % HW
            y = r // W
            xx = r % W
            p = b * PB + (y + R) * Wp + xx + R
            pad_s[pl.ds(p, 1), :] = h_s[pl.ds(i, 1), :]
            return c

        lax.fori_loop(0, N, scat, 0)

        def conv_blk(m, c):
            b = m // nblk
            g0 = (m % nblk) * BLK

            def tap(tp, acc):
                o = (tp // K) * Wp + (tp % K)
                src = pad_s[pl.ds(b * PB + g0 + o, BLK), :]
                wk = w_r[tp]
                return acc + jnp.dot(src, wk, preferred_element_type=f32)

            accv = lax.fori_loop(0, K * K, tap, jnp.zeros((BLK, C), f32))
            acc_s[pl.ds(b * ACC_B + g0, BLK), :] = accv
            return c

        lax.fori_loop(0, B * nblk, conv_blk, 0)

        def gath(i, c):
            v = idx_s[i]
            b = v // HW
            r = v % HW
            y = r // W
            xx = r % W
            g = b * ACC_B + y * Wp + xx
            hc_s[pl.ds(i, 1), :] = acc_s[pl.ds(g, 1), :]
            return c

        lax.fori_loop(0, N, gath, 0)

        ln2g = ln2g_r[...]
        ln2b = ln2b_r[...]
        m1w = m1w_r[...]
        m1b = m1b_r[...]
        m2w = m2w_r[...]
        m2b = m2b_r[...]
        for rb in range(nrb):
            sl = pl.ds(rb * RB, RB)
            bi = (rb * RB) // NB
            x1 = x_r[sl, :] + hc_s[sl, :] / norm_r[sl, :]
            f2 = _ln(x1, ln2g, ln2b)
            h2 = f2 * (1.0 + tt2[bi:bi + 1, :C]) + tt2[bi:bi + 1, C:]
            hid = _gelu(jnp.dot(h2, m1w, preferred_element_type=f32) + m1b)
            o = jnp.dot(hid, m2w, preferred_element_type=f32) + m2b
            out_r[sl, :] = x1 + o

    vspec = pl.BlockSpec(memory_space=pltpu.VMEM)
    smem = pl.BlockSpec(memory_space=pltpu.SMEM)
    vmem = pltpu.VMEM
    f = pl.pallas_call(
        body,
        out_shape=jax.ShapeDtypeStruct((N, C), jnp.float32),
        in_specs=[smem] + [vspec] * 16,
        out_specs=vspec,
        scratch_shapes=[
            vmem((B * PB, C), jnp.float32),
            vmem((B * ACC_B, C), jnp.float32),
            vmem((N, C), jnp.float32),
            vmem((N, C), jnp.float32),
        ],
        interpret=interpret,
    )
    return f(flat_idx.astype(jnp.int32), x, t, norm, w49,
             ln1_g.reshape(1, C), ln1_b.reshape(1, C),
             ln2_g.reshape(1, C), ln2_b.reshape(1, C),
             tmlp1_w, tmlp1_b.reshape(1, 2 * C),
             tmlp2_w, tmlp2_b.reshape(1, 2 * C),
             mlp_w1, mlp_b1.reshape(1, 2 * C),
             mlp_w2, mlp_b2.reshape(1, C))


def kernel(x, t, norm, flat_idx, conv_w, ln1_g, ln1_b, ln2_g, ln2_b,
           tmlp1_w, tmlp1_b, tmlp2_w, tmlp2_b, mlp_w1, mlp_b1, mlp_w2, mlp_b2):
    return _impl(64, 64, x, t, norm, flat_idx, conv_w, ln1_g, ln1_b,
                 ln2_g, ln2_b, tmlp1_w, tmlp1_b, tmlp2_w, tmlp2_b,
                 mlp_w1, mlp_b1, mlp_w2, mlp_b2)
